# baseline (speedup 1.0000x reference)
import functools

import jax
import jax.numpy as jnp
from jax import lax
from jax.experimental import pallas as pl
from jax.experimental.pallas import tpu as pltpu
from jax.experimental.pallas import tpu_sc as plsc

NC = 2
NS = 16
NW = NC * NS
CHUNK = 128
DEGW = 16


def _ceil_div(a, b):
    return (a + b - 1) // b


def _sc_mesh():
    return plsc.VectorSubcoreMesh(core_axis_name="c", subcore_axis_name="s")


def _sc_params():
    return pltpu.CompilerParams(use_tc_tiling_on_sc=False)


@functools.lru_cache(maxsize=None)
def _deg_fn(N, E):
    assert E % NW == 0 and N % NS == 0
    epw = E // NW
    full = epw // CHUNK
    tail = epw % CHUNK
    nb = full // 3
    rem = full % 3
    stripe = (N // NS) & ~7
    last = N - (NS - 1) * stripe
    zlen = (last + 15) & ~15

    def body(ei_hbm, out_hbm, idx_v, ones_v, zero_v, acc_sh, sem_s):
        cid = lax.axis_index("c")
        sid = lax.axis_index("s")
        wid = cid * NS + sid

        @pl.loop(0, CHUNK, step=16)
        def _fill(i):
            ones_v[pl.ds(i, 16)] = jnp.ones((16,), jnp.float32)

        @pl.loop(0, zlen, step=16)
        def _zfill(i):
            zero_v[pl.ds(i, 16)] = jnp.zeros((16,), jnp.float32)

        @pl.when(sid < NS - 1)
        def _():
            pltpu.sync_copy(zero_v.at[pl.ds(0, stripe)],
                            acc_sh.at[pl.ds(sid * stripe, stripe)])

        @pl.when(sid == NS - 1)
        def _():
            pltpu.sync_copy(zero_v.at[pl.ds(0, last)],
                            acc_sh.at[pl.ds(sid * stripe, last)])

        pltpu.sync_copy(ei_hbm.at[1, pl.ds(wid * epw, epw)], idx_v.at[0])
        plsc.subcore_barrier()

        def fire(j):
            pltpu.async_copy(
                ones_v, acc_sh.at[idx_v.at[0, pl.ds(j * CHUNK, CHUNK)]],
                sem_s, add=True)

        def drain(j):
            pltpu.make_async_copy(
                ones_v, acc_sh.at[idx_v.at[0, pl.ds(j * CHUNK, CHUNK)]],
                sem_s).wait()

        @pl.loop(0, nb)
        def _edges(t):
            j = t * 3
            fire(j)
            fire(j + 1)
            fire(j + 2)

            @pl.when(t >= 1)
            def _():
                drain(j - 3)
                drain(j - 2)
                drain(j - 1)

        if nb > 0:
            drain(nb * 3 - 3)
            drain(nb * 3 - 2)
            drain(nb * 3 - 1)
        for u in range(rem):
            j = nb * 3 + u
            pltpu.sync_copy(
                ones_v, acc_sh.at[idx_v.at[0, pl.ds(j * CHUNK, CHUNK)]],
                add=True)
        if tail:
            pltpu.sync_copy(
                ones_v.at[pl.ds(0, tail)],
                acc_sh.at[idx_v.at[0, pl.ds(full * CHUNK, tail)]],
                add=True)

        plsc.subcore_barrier()

        @pl.when(sid < NS - 1)
        def _():
            pltpu.sync_copy(acc_sh.at[pl.ds(sid * stripe, stripe)],
                            out_hbm.at[cid, pl.ds(sid * stripe, stripe)])

        @pl.when(sid == NS - 1)
        def _():
            pltpu.sync_copy(acc_sh.at[pl.ds(sid * stripe, last)],
                            out_hbm.at[cid, pl.ds(sid * stripe, last)])

    return pl.kernel(
        body,
        out_type=jax.ShapeDtypeStruct((NC, N), jnp.float32),
        mesh=_sc_mesh(),
        compiler_params=_sc_params(),
        scratch_types=[
            pltpu.VMEM((1, epw), jnp.int32),
            pltpu.VMEM((CHUNK,), jnp.float32),
            pltpu.VMEM((zlen,), jnp.float32),
            pltpu.VMEM_SHARED((N,), jnp.float32),
            pltpu.SemaphoreType.DMA,
        ],
    )


@functools.lru_cache(maxsize=None)
def _agg_fn(N, E, F):
    assert E % NW == 0 and N % NS == 0 and F % 8 == 0
    epw = E // NW
    full = epw // CHUNK
    tail = epw % CHUNK
    nb = full // 4
    rem = full % 4
    loopn = nb * 4
    rpw = N // NS
    nz = _ceil_div(rpw, CHUNK)
    assert rpw % nz == 0
    zb = rpw // nz

    def body(src_hbm, ei_hbm, out_hbm, idx_v, rows_v, acc_sh,
             sg0, sg1, sg2, sg3, ss0, ss1, ss2, ss3):
        cid = lax.axis_index("c")
        sid = lax.axis_index("s")
        wid = cid * NS + sid
        sem_g = (sg0, sg1, sg2, sg3)
        sem_s = (ss0, ss1, ss2, ss3)

        zoffs = list(range(0, F - 15, 16))
        if F % 16:
            zoffs.append(F - 16)

        @pl.loop(0, CHUNK)
        def _fill(i):
            for j in zoffs:
                rows_v[0, i, pl.ds(j, 16)] = jnp.zeros((16,), jnp.float32)

        @pl.loop(0, nz)
        def _zero(k):
            pltpu.sync_copy(rows_v.at[0, pl.ds(0, zb)],
                            acc_sh.at[pl.ds(sid * rpw + k * zb, zb)])

        pltpu.sync_copy(ei_hbm.at[:, pl.ds(wid * epw, epw)], idx_v)
        plsc.subcore_barrier()

        def gidx(j):
            return idx_v.at[0, pl.ds(j * CHUNK, CHUNK)]

        def sidx(j):
            return idx_v.at[1, pl.ds(j * CHUNK, CHUNK)]

        def fire_g(j, b):
            pltpu.async_copy(src_hbm.at[gidx(j)], rows_v.at[b], sem_g[b])

        def wait_g(j, b):
            pltpu.make_async_copy(src_hbm.at[gidx(j)], rows_v.at[b],
                                  sem_g[b]).wait()

        def fire_s(j, b):
            pltpu.async_copy(rows_v.at[b], acc_sh.at[sidx(j)], sem_s[b],
                             add=True)

        def wait_s(j, b):
            pltpu.make_async_copy(rows_v.at[b], acc_sh.at[sidx(j)],
                                  sem_s[b]).wait()

        if loopn > 0:
            fire_g(0, 0)
        if loopn > 1:
            fire_g(1, 1)

        @pl.loop(0, nb)
        def _edges(t):
            for u in range(4):
                j = t * 4 + u
                nx2 = (u + 2) % 4

                if u < 2:
                    @pl.when(j >= 2)
                    def _():
                        wait_s(j - 2, nx2)
                else:
                    wait_s(j - 2, nx2)

                if (loopn - 4 + u) + 2 < loopn:
                    fire_g(j + 2, nx2)
                else:
                    @pl.when(j + 2 < loopn)
                    def _():
                        fire_g(j + 2, nx2)
                wait_g(j, u)
                fire_s(j, u)

        if loopn > 1:
            wait_s(loopn - 2, (loopn - 2) % 4)
        if loopn > 0:
            wait_s(loopn - 1, (loopn - 1) % 4)
        for u in range(rem):
            j = loopn + u
            pltpu.sync_copy(src_hbm.at[gidx(j)], rows_v.at[0])
            pltpu.sync_copy(rows_v.at[0], acc_sh.at[sidx(j)], add=True)
        if tail:
            pltpu.sync_copy(
                src_hbm.at[idx_v.at[0, pl.ds(full * CHUNK, tail)]],
                rows_v.at[0, pl.ds(0, tail)])
            pltpu.sync_copy(
                rows_v.at[0, pl.ds(0, tail)],
                acc_sh.at[idx_v.at[1, pl.ds(full * CHUNK, tail)]],
                add=True)

        plsc.subcore_barrier()
        pltpu.sync_copy(acc_sh.at[pl.ds(sid * rpw, rpw)],
                        out_hbm.at[cid, pl.ds(sid * rpw, rpw)])

    return pl.kernel(
        body,
        out_type=jax.ShapeDtypeStruct((NC, N, F), jnp.float32),
        mesh=_sc_mesh(),
        compiler_params=_sc_params(),
        scratch_types=[
            pltpu.VMEM((2, epw), jnp.int32),
            pltpu.VMEM((4, CHUNK, F), jnp.float32),
            pltpu.VMEM_SHARED((N, F), jnp.float32),
            pltpu.SemaphoreType.DMA,
            pltpu.SemaphoreType.DMA,
            pltpu.SemaphoreType.DMA,
            pltpu.SemaphoreType.DMA,
            pltpu.SemaphoreType.DMA,
            pltpu.SemaphoreType.DMA,
            pltpu.SemaphoreType.DMA,
            pltpu.SemaphoreType.DMA,
        ],
    )


def _dinv(degp_ref):
    d = degp_ref[:, 0:1] + degp_ref[:, 1:2] + 1.0
    return lax.rsqrt(d)


def _matmul_t(a, w_ref):
    return lax.dot_general(
        a, w_ref[...], dimension_numbers=(((1,), (1,)), ((), ())),
        preferred_element_type=jnp.float32, precision=lax.Precision.DEFAULT)


def _tc_mm1(x, W1, bn):
    N, D = x.shape
    H = W1.shape[0]

    def body(x_ref, w1_ref, o_ref):
        o_ref[...] = _matmul_t(x_ref[...], w1_ref)

    grid = (N // bn,)
    return pl.pallas_call(
        body,
        grid=grid,
        in_specs=[
            pl.BlockSpec((bn, D), lambda i: (i, 0)),
            pl.BlockSpec((H, D), lambda i: (0, 0)),
        ],
        out_specs=pl.BlockSpec((bn, H), lambda i: (i, 0)),
        out_shape=jax.ShapeDtypeStruct((N, H), jnp.float32),
    )(x, W1)


def _tc_scale(xw1, degt, bn):
    N, H = xw1.shape

    def body(xw_ref, degp_ref, o_ref):
        o_ref[...] = xw_ref[...] * _dinv(degp_ref)

    grid = (N // bn,)
    return pl.pallas_call(
        body,
        grid=grid,
        in_specs=[
            pl.BlockSpec((bn, H), lambda i: (i, 0)),
            pl.BlockSpec((bn, 2), lambda i: (i, 0)),
        ],
        out_specs=pl.BlockSpec((bn, H), lambda i: (i, 0)),
        out_shape=jax.ShapeDtypeStruct((N, H), jnp.float32),
    )(xw1, degt)


def _tc_mid(agg1, xs1, degt, W2, b1, bn, Fp):
    _, N, H = agg1.shape
    C = W2.shape[0]

    def body(a_ref, xs1_ref, degp_ref, w2_ref, b1_ref, o_ref):
        dinv = _dinv(degp_ref)
        s = a_ref[0] + a_ref[1] + xs1_ref[...]
        h = jnp.maximum(dinv * s + b1_ref[...], 0.0)
        xs2 = _matmul_t(h, w2_ref) * dinv
        if Fp > C:
            xs2 = jnp.concatenate(
                [xs2, jnp.zeros((xs2.shape[0], Fp - C), jnp.float32)], axis=1)
        o_ref[...] = xs2

    grid = (N // bn,)
    return pl.pallas_call(
        body,
        grid=grid,
        in_specs=[
            pl.BlockSpec((NC, bn, H), lambda i: (0, i, 0)),
            pl.BlockSpec((bn, H), lambda i: (i, 0)),
            pl.BlockSpec((bn, 2), lambda i: (i, 0)),
            pl.BlockSpec((C, H), lambda i: (0, 0)),
            pl.BlockSpec((1, H), lambda i: (0, 0)),
        ],
        out_specs=pl.BlockSpec((bn, Fp), lambda i: (i, 0)),
        out_shape=jax.ShapeDtypeStruct((N, Fp), jnp.float32),
    )(agg1, xs1, degt, W2, b1)


def _tc_final(agg2, xs2, degt, b2, bn):
    _, N, Fp = agg2.shape
    C = b2.shape[1]

    def body(a_ref, xs2_ref, degp_ref, b2_ref, o_ref):
        dinv = _dinv(degp_ref)
        s = a_ref[0] + a_ref[1] + xs2_ref[...]
        o_ref[...] = dinv * s[:, 0:C] + b2_ref[...]

    grid = (N // bn,)
    return pl.pallas_call(
        body,
        grid=grid,
        in_specs=[
            pl.BlockSpec((NC, bn, Fp), lambda i: (0, i, 0)),
            pl.BlockSpec((bn, Fp), lambda i: (i, 0)),
            pl.BlockSpec((bn, 2), lambda i: (i, 0)),
            pl.BlockSpec((1, C), lambda i: (0, 0)),
        ],
        out_specs=pl.BlockSpec((bn, C), lambda i: (i, 0)),
        out_shape=jax.ShapeDtypeStruct((N, C), jnp.float32),
    )(agg2, xs2, degt, b2)


def kernel(x, edge_index, W1, b1, W2, b2):
    N, D = x.shape
    H = W1.shape[0]
    C = W2.shape[0]
    E = edge_index.shape[1]
    Fp = _ceil_div(C, 8) * 8
    bn = 1000

    xw1 = _tc_mm1(x, W1, bn)
    degp = _deg_fn(N, E)(edge_index)
    degt = degp.T
    xs1 = _tc_scale(xw1, degt, bn)
    agg1 = _agg_fn(N, E, H)(xs1, edge_index)
    xs2 = _tc_mid(agg1, xs1, degt, W2, b1.reshape(1, H), bn, Fp)
    agg2 = _agg_fn(N, E, Fp)(xs2, edge_index)
    out = _tc_final(agg2, xs2, degt, b2.reshape(1, C), bn)
    return out


# speedup vs baseline: 1.0375x; 1.0375x over previous
import functools

import jax
import jax.numpy as jnp
from jax import lax
from jax.experimental import pallas as pl
from jax.experimental.pallas import tpu as pltpu
from jax.experimental.pallas import tpu_sc as plsc

NC = 2
NS = 16
NW = NC * NS
CHUNK = 128
DEGW = 16


def _ceil_div(a, b):
    return (a + b - 1) // b


def _sc_mesh():
    return plsc.VectorSubcoreMesh(core_axis_name="c", subcore_axis_name="s")


def _sc_params():
    return pltpu.CompilerParams(use_tc_tiling_on_sc=False)


@functools.lru_cache(maxsize=None)
def _deg_fn(N, E):
    assert E % NW == 0 and N % NS == 0
    epw = E // NW
    full = epw // CHUNK
    tail = epw % CHUNK
    nb = full // 3
    rem = full % 3
    stripe = (N // NS) & ~7
    last = N - (NS - 1) * stripe
    zlen = (last + 15) & ~15

    def body(ei_hbm, out_hbm, idx_v, ones_v, zero_v, acc_sh, sem_s):
        cid = lax.axis_index("c")
        sid = lax.axis_index("s")
        wid = cid * NS + sid

        @pl.loop(0, CHUNK, step=16)
        def _fill(i):
            ones_v[pl.ds(i, 16)] = jnp.ones((16,), jnp.float32)

        @pl.loop(0, zlen, step=16)
        def _zfill(i):
            zero_v[pl.ds(i, 16)] = jnp.zeros((16,), jnp.float32)

        @pl.when(sid < NS - 1)
        def _():
            pltpu.sync_copy(zero_v.at[pl.ds(0, stripe)],
                            acc_sh.at[pl.ds(sid * stripe, stripe)])

        @pl.when(sid == NS - 1)
        def _():
            pltpu.sync_copy(zero_v.at[pl.ds(0, last)],
                            acc_sh.at[pl.ds(sid * stripe, last)])

        pltpu.sync_copy(ei_hbm.at[1, pl.ds(wid * epw, epw)], idx_v.at[0])
        plsc.subcore_barrier()

        def fire(j):
            pltpu.async_copy(
                ones_v, acc_sh.at[idx_v.at[0, pl.ds(j * CHUNK, CHUNK)]],
                sem_s, add=True)

        def drain(j):
            pltpu.make_async_copy(
                ones_v, acc_sh.at[idx_v.at[0, pl.ds(j * CHUNK, CHUNK)]],
                sem_s).wait()

        @pl.loop(0, nb)
        def _edges(t):
            j = t * 3
            fire(j)
            fire(j + 1)
            fire(j + 2)

            @pl.when(t >= 1)
            def _():
                drain(j - 3)
                drain(j - 2)
                drain(j - 1)

        if nb > 0:
            drain(nb * 3 - 3)
            drain(nb * 3 - 2)
            drain(nb * 3 - 1)
        for u in range(rem):
            j = nb * 3 + u
            pltpu.sync_copy(
                ones_v, acc_sh.at[idx_v.at[0, pl.ds(j * CHUNK, CHUNK)]],
                add=True)
        if tail:
            pltpu.sync_copy(
                ones_v.at[pl.ds(0, tail)],
                acc_sh.at[idx_v.at[0, pl.ds(full * CHUNK, tail)]],
                add=True)

        plsc.subcore_barrier()

        @pl.when(sid < NS - 1)
        def _():
            pltpu.sync_copy(acc_sh.at[pl.ds(sid * stripe, stripe)],
                            out_hbm.at[cid, pl.ds(sid * stripe, stripe)])

        @pl.when(sid == NS - 1)
        def _():
            pltpu.sync_copy(acc_sh.at[pl.ds(sid * stripe, last)],
                            out_hbm.at[cid, pl.ds(sid * stripe, last)])

    return pl.kernel(
        body,
        out_type=jax.ShapeDtypeStruct((NC, N), jnp.float32),
        mesh=_sc_mesh(),
        compiler_params=_sc_params(),
        scratch_types=[
            pltpu.VMEM((1, epw), jnp.int32),
            pltpu.VMEM((CHUNK,), jnp.float32),
            pltpu.VMEM((zlen,), jnp.float32),
            pltpu.VMEM_SHARED((N,), jnp.float32),
            pltpu.SemaphoreType.DMA,
        ],
    )


@functools.lru_cache(maxsize=None)
def _agg_fn(N, E, F):
    assert E % NW == 0 and N % NS == 0 and F % 8 == 0
    epw = E // NW
    full = epw // CHUNK
    tail = epw % CHUNK
    nb = full // 4
    rem = full % 4
    loopn = nb * 4
    rpw = N // NS
    nz = _ceil_div(rpw, CHUNK)
    assert rpw % nz == 0
    zb = rpw // nz

    def body(src_hbm, ei_hbm, out_hbm, idx_v, rows_v, acc_sh,
             sg0, sg1, sg2, sg3, ss0, ss1, ss2, ss3):
        cid = lax.axis_index("c")
        sid = lax.axis_index("s")
        wid = cid * NS + sid
        sem_g = (sg0, sg1, sg2, sg3)
        sem_s = (ss0, ss1, ss2, ss3)

        zoffs = list(range(0, F - 15, 16))
        if F % 16:
            zoffs.append(F - 16)

        @pl.loop(0, CHUNK)
        def _fill(i):
            for j in zoffs:
                rows_v[0, i, pl.ds(j, 16)] = jnp.zeros((16,), jnp.float32)

        @pl.loop(0, nz)
        def _zero(k):
            pltpu.sync_copy(rows_v.at[0, pl.ds(0, zb)],
                            acc_sh.at[pl.ds(sid * rpw + k * zb, zb)])

        pltpu.sync_copy(ei_hbm.at[:, pl.ds(wid * epw, epw)], idx_v)
        plsc.subcore_barrier()

        def gidx(j):
            return idx_v.at[0, pl.ds(j * CHUNK, CHUNK)]

        def sidx(j):
            return idx_v.at[1, pl.ds(j * CHUNK, CHUNK)]

        def fire_g(j, b):
            pltpu.async_copy(src_hbm.at[gidx(j)], rows_v.at[b], sem_g[b])

        def wait_g(j, b):
            pltpu.make_async_copy(src_hbm.at[gidx(j)], rows_v.at[b],
                                  sem_g[b]).wait()

        def fire_s(j, b):
            pltpu.async_copy(rows_v.at[b], acc_sh.at[sidx(j)], sem_s[b],
                             add=True)

        def wait_s(j, b):
            pltpu.make_async_copy(rows_v.at[b], acc_sh.at[sidx(j)],
                                  sem_s[b]).wait()

        if loopn > 0:
            fire_g(0, 0)
        if loopn > 1:
            fire_g(1, 1)

        @pl.loop(0, nb)
        def _edges(t):
            for u in range(4):
                j = t * 4 + u
                nx2 = (u + 2) % 4

                if u < 2:
                    @pl.when(j >= 2)
                    def _():
                        wait_s(j - 2, nx2)
                else:
                    wait_s(j - 2, nx2)

                if (loopn - 4 + u) + 2 < loopn:
                    fire_g(j + 2, nx2)
                else:
                    @pl.when(j + 2 < loopn)
                    def _():
                        fire_g(j + 2, nx2)
                wait_g(j, u)
                fire_s(j, u)

        if loopn > 1:
            wait_s(loopn - 2, (loopn - 2) % 4)
        if loopn > 0:
            wait_s(loopn - 1, (loopn - 1) % 4)
        for u in range(rem):
            j = loopn + u
            pltpu.sync_copy(src_hbm.at[gidx(j)], rows_v.at[0])
            pltpu.sync_copy(rows_v.at[0], acc_sh.at[sidx(j)], add=True)
        if tail:
            pltpu.sync_copy(
                src_hbm.at[idx_v.at[0, pl.ds(full * CHUNK, tail)]],
                rows_v.at[0, pl.ds(0, tail)])
            pltpu.sync_copy(
                rows_v.at[0, pl.ds(0, tail)],
                acc_sh.at[idx_v.at[1, pl.ds(full * CHUNK, tail)]],
                add=True)

        plsc.subcore_barrier()
        pltpu.sync_copy(acc_sh.at[pl.ds(sid * rpw, rpw)],
                        out_hbm.at[cid, pl.ds(sid * rpw, rpw)])

    return pl.kernel(
        body,
        out_type=jax.ShapeDtypeStruct((NC, N, F), jnp.float32),
        mesh=_sc_mesh(),
        compiler_params=_sc_params(),
        scratch_types=[
            pltpu.VMEM((2, epw), jnp.int32),
            pltpu.VMEM((4, CHUNK, F), jnp.float32),
            pltpu.VMEM_SHARED((N, F), jnp.float32),
            pltpu.SemaphoreType.DMA,
            pltpu.SemaphoreType.DMA,
            pltpu.SemaphoreType.DMA,
            pltpu.SemaphoreType.DMA,
            pltpu.SemaphoreType.DMA,
            pltpu.SemaphoreType.DMA,
            pltpu.SemaphoreType.DMA,
            pltpu.SemaphoreType.DMA,
        ],
    )


def _dinv(degp_ref):
    d = degp_ref[:, 0:1] + degp_ref[:, 1:2] + 1.0
    return lax.rsqrt(d)


def _matmul_t(a, w_ref):
    return lax.dot_general(
        a, w_ref[...], dimension_numbers=(((1,), (1,)), ((), ())),
        preferred_element_type=jnp.float32, precision=lax.Precision.DEFAULT)


def _tc_mm1(x, W1, bn):
    N, D = x.shape
    H = W1.shape[0]

    def body(x_ref, w1_ref, o_ref):
        o_ref[...] = _matmul_t(x_ref[...], w1_ref)

    grid = (N // bn,)
    return pl.pallas_call(
        body,
        grid=grid,
        in_specs=[
            pl.BlockSpec((bn, D), lambda i: (i, 0)),
            pl.BlockSpec((H, D), lambda i: (0, 0)),
        ],
        out_specs=pl.BlockSpec((bn, H), lambda i: (i, 0)),
        out_shape=jax.ShapeDtypeStruct((N, H), jnp.float32),
    )(x, W1)


def _tc_scale(xw1, degt, bn):
    N, H = xw1.shape

    def body(xw_ref, degp_ref, o_ref):
        o_ref[...] = xw_ref[...] * _dinv(degp_ref)

    grid = (N // bn,)
    return pl.pallas_call(
        body,
        grid=grid,
        in_specs=[
            pl.BlockSpec((bn, H), lambda i: (i, 0)),
            pl.BlockSpec((bn, 2), lambda i: (i, 0)),
        ],
        out_specs=pl.BlockSpec((bn, H), lambda i: (i, 0)),
        out_shape=jax.ShapeDtypeStruct((N, H), jnp.float32),
    )(xw1, degt)


def _tc_mid(agg1, xs1, degt, W2, b1, bn, Fp):
    _, N, H = agg1.shape
    C = W2.shape[0]

    def body(a_ref, xs1_ref, degp_ref, w2_ref, b1_ref, o_ref):
        dinv = _dinv(degp_ref)
        s = a_ref[0] + a_ref[1] + xs1_ref[...]
        h = jnp.maximum(dinv * s + b1_ref[...], 0.0)
        xs2 = _matmul_t(h, w2_ref) * dinv
        if Fp > C:
            xs2 = jnp.concatenate(
                [xs2, jnp.zeros((xs2.shape[0], Fp - C), jnp.float32)], axis=1)
        o_ref[...] = xs2

    grid = (N // bn,)
    return pl.pallas_call(
        body,
        grid=grid,
        in_specs=[
            pl.BlockSpec((NC, bn, H), lambda i: (0, i, 0)),
            pl.BlockSpec((bn, H), lambda i: (i, 0)),
            pl.BlockSpec((bn, 2), lambda i: (i, 0)),
            pl.BlockSpec((C, H), lambda i: (0, 0)),
            pl.BlockSpec((1, H), lambda i: (0, 0)),
        ],
        out_specs=pl.BlockSpec((bn, Fp), lambda i: (i, 0)),
        out_shape=jax.ShapeDtypeStruct((N, Fp), jnp.float32),
    )(agg1, xs1, degt, W2, b1)


def _tc_final(agg2, xs2, degt, b2, bn):
    _, N, Fp = agg2.shape
    C = b2.shape[1]

    def body(a_ref, xs2_ref, degp_ref, b2_ref, o_ref):
        dinv = _dinv(degp_ref)
        s = a_ref[0] + a_ref[1] + xs2_ref[...]
        o_ref[...] = dinv * s[:, 0:C] + b2_ref[...]

    grid = (N // bn,)
    return pl.pallas_call(
        body,
        grid=grid,
        in_specs=[
            pl.BlockSpec((NC, bn, Fp), lambda i: (0, i, 0)),
            pl.BlockSpec((bn, Fp), lambda i: (i, 0)),
            pl.BlockSpec((bn, 2), lambda i: (i, 0)),
            pl.BlockSpec((1, C), lambda i: (0, 0)),
        ],
        out_specs=pl.BlockSpec((bn, C), lambda i: (i, 0)),
        out_shape=jax.ShapeDtypeStruct((N, C), jnp.float32),
    )(agg2, xs2, degt, b2)


def kernel(x, edge_index, W1, b1, W2, b2):
    N, D = x.shape
    H = W1.shape[0]
    C = W2.shape[0]
    E = edge_index.shape[1]
    Fp = _ceil_div(C, 8) * 8
    bn = 2000

    xw1 = _tc_mm1(x, W1, bn)
    degp = _deg_fn(N, E)(edge_index)
    degt = degp.T
    xs1 = _tc_scale(xw1, degt, bn)
    agg1 = _agg_fn(N, E, H)(xs1, edge_index)
    xs2 = _tc_mid(agg1, xs1, degt, W2, b1.reshape(1, H), bn, Fp)
    agg2 = _agg_fn(N, E, Fp)(xs2, edge_index)
    out = _tc_final(agg2, xs2, degt, b2.reshape(1, C), bn)
    return out


# speedup vs baseline: 1.0566x; 1.0183x over previous
import functools

import jax
import jax.numpy as jnp
from jax import lax
from jax.experimental import pallas as pl
from jax.experimental.pallas import tpu as pltpu
from jax.experimental.pallas import tpu_sc as plsc

NC = 2
NS = 16
NW = NC * NS
CHUNK = 128
DEGW = 16


def _ceil_div(a, b):
    return (a + b - 1) // b


def _sc_mesh():
    return plsc.VectorSubcoreMesh(core_axis_name="c", subcore_axis_name="s")


def _sc_params():
    return pltpu.CompilerParams(use_tc_tiling_on_sc=False)


@functools.lru_cache(maxsize=None)
def _deg_fn(N, E):
    assert E % NW == 0 and N % NS == 0
    epw = E // NW
    full = epw // CHUNK
    tail = epw % CHUNK
    nb = full // 3
    rem = full % 3
    stripe = (N // NS) & ~7
    last = N - (NS - 1) * stripe
    zlen = (last + 15) & ~15

    def body(ei_hbm, out_hbm, idx_v, ones_v, zero_v, acc_sh, sem_s):
        cid = lax.axis_index("c")
        sid = lax.axis_index("s")
        wid = cid * NS + sid

        @pl.loop(0, CHUNK, step=16)
        def _fill(i):
            ones_v[pl.ds(i, 16)] = jnp.ones((16,), jnp.float32)

        @pl.loop(0, zlen, step=16)
        def _zfill(i):
            zero_v[pl.ds(i, 16)] = jnp.zeros((16,), jnp.float32)

        @pl.when(sid < NS - 1)
        def _():
            pltpu.sync_copy(zero_v.at[pl.ds(0, stripe)],
                            acc_sh.at[pl.ds(sid * stripe, stripe)])

        @pl.when(sid == NS - 1)
        def _():
            pltpu.sync_copy(zero_v.at[pl.ds(0, last)],
                            acc_sh.at[pl.ds(sid * stripe, last)])

        pltpu.sync_copy(ei_hbm.at[1, pl.ds(wid * epw, epw)], idx_v.at[0])
        plsc.subcore_barrier()

        def fire(j):
            pltpu.async_copy(
                ones_v, acc_sh.at[idx_v.at[0, pl.ds(j * CHUNK, CHUNK)]],
                sem_s, add=True)

        def drain(j):
            pltpu.make_async_copy(
                ones_v, acc_sh.at[idx_v.at[0, pl.ds(j * CHUNK, CHUNK)]],
                sem_s).wait()

        @pl.loop(0, nb)
        def _edges(t):
            j = t * 3
            fire(j)
            fire(j + 1)
            fire(j + 2)

            @pl.when(t >= 1)
            def _():
                drain(j - 3)
                drain(j - 2)
                drain(j - 1)

        if nb > 0:
            drain(nb * 3 - 3)
            drain(nb * 3 - 2)
            drain(nb * 3 - 1)
        for u in range(rem):
            j = nb * 3 + u
            pltpu.sync_copy(
                ones_v, acc_sh.at[idx_v.at[0, pl.ds(j * CHUNK, CHUNK)]],
                add=True)
        if tail:
            pltpu.sync_copy(
                ones_v.at[pl.ds(0, tail)],
                acc_sh.at[idx_v.at[0, pl.ds(full * CHUNK, tail)]],
                add=True)

        plsc.subcore_barrier()

        @pl.when(sid < NS - 1)
        def _():
            pltpu.sync_copy(acc_sh.at[pl.ds(sid * stripe, stripe)],
                            out_hbm.at[cid, pl.ds(sid * stripe, stripe)])

        @pl.when(sid == NS - 1)
        def _():
            pltpu.sync_copy(acc_sh.at[pl.ds(sid * stripe, last)],
                            out_hbm.at[cid, pl.ds(sid * stripe, last)])

    return pl.kernel(
        body,
        out_type=jax.ShapeDtypeStruct((NC, N), jnp.float32),
        mesh=_sc_mesh(),
        compiler_params=_sc_params(),
        scratch_types=[
            pltpu.VMEM((1, epw), jnp.int32),
            pltpu.VMEM((CHUNK,), jnp.float32),
            pltpu.VMEM((zlen,), jnp.float32),
            pltpu.VMEM_SHARED((N,), jnp.float32),
            pltpu.SemaphoreType.DMA,
        ],
    )


@functools.lru_cache(maxsize=None)
def _agg_fn(N, E, F):
    assert E % NW == 0 and N % NS == 0 and F % 8 == 0
    epw = E // NW
    full = epw // CHUNK
    tail = epw % CHUNK
    nb = full // 4
    rem = full % 4
    loopn = nb * 4
    rpw = N // NS
    nz = _ceil_div(rpw, CHUNK)
    assert rpw % nz == 0
    zb = rpw // nz

    def body(src_hbm, ei_hbm, out_hbm, idx_v, rows_v, acc_sh,
             sg0, sg1, sg2, sg3, ss0, ss1, ss2, ss3):
        cid = lax.axis_index("c")
        sid = lax.axis_index("s")
        wid = cid * NS + sid
        sem_g = (sg0, sg1, sg2, sg3)
        sem_s = (ss0, ss1, ss2, ss3)

        pltpu.async_copy(ei_hbm.at[:, pl.ds(wid * epw, epw)], idx_v, sg0)

        zoffs = list(range(0, F - 15, 16))
        if F % 16:
            zoffs.append(F - 16)

        @pl.loop(0, CHUNK)
        def _fill(i):
            for j in zoffs:
                rows_v[0, i, pl.ds(j, 16)] = jnp.zeros((16,), jnp.float32)

        @pl.loop(0, nz)
        def _zero(k):
            pltpu.sync_copy(rows_v.at[0, pl.ds(0, zb)],
                            acc_sh.at[pl.ds(sid * rpw + k * zb, zb)])

        pltpu.make_async_copy(ei_hbm.at[:, pl.ds(wid * epw, epw)], idx_v,
                              sg0).wait()
        plsc.subcore_barrier()

        def gidx(j):
            return idx_v.at[0, pl.ds(j * CHUNK, CHUNK)]

        def sidx(j):
            return idx_v.at[1, pl.ds(j * CHUNK, CHUNK)]

        def fire_g(j, b):
            pltpu.async_copy(src_hbm.at[gidx(j)], rows_v.at[b], sem_g[b])

        def wait_g(j, b):
            pltpu.make_async_copy(src_hbm.at[gidx(j)], rows_v.at[b],
                                  sem_g[b]).wait()

        def fire_s(j, b):
            pltpu.async_copy(rows_v.at[b], acc_sh.at[sidx(j)], sem_s[b],
                             add=True)

        def wait_s(j, b):
            pltpu.make_async_copy(rows_v.at[b], acc_sh.at[sidx(j)],
                                  sem_s[b]).wait()

        if loopn > 0:
            fire_g(0, 0)
        if loopn > 1:
            fire_g(1, 1)

        @pl.loop(0, nb)
        def _edges(t):
            for u in range(4):
                j = t * 4 + u
                nx2 = (u + 2) % 4

                if u < 2:
                    @pl.when(j >= 2)
                    def _():
                        wait_s(j - 2, nx2)
                else:
                    wait_s(j - 2, nx2)

                if (loopn - 4 + u) + 2 < loopn:
                    fire_g(j + 2, nx2)
                else:
                    @pl.when(j + 2 < loopn)
                    def _():
                        fire_g(j + 2, nx2)
                wait_g(j, u)
                fire_s(j, u)

        if loopn > 1:
            wait_s(loopn - 2, (loopn - 2) % 4)
        if loopn > 0:
            wait_s(loopn - 1, (loopn - 1) % 4)
        for u in range(rem):
            j = loopn + u
            pltpu.sync_copy(src_hbm.at[gidx(j)], rows_v.at[0])
            pltpu.sync_copy(rows_v.at[0], acc_sh.at[sidx(j)], add=True)
        if tail:
            pltpu.sync_copy(
                src_hbm.at[idx_v.at[0, pl.ds(full * CHUNK, tail)]],
                rows_v.at[0, pl.ds(0, tail)])
            pltpu.sync_copy(
                rows_v.at[0, pl.ds(0, tail)],
                acc_sh.at[idx_v.at[1, pl.ds(full * CHUNK, tail)]],
                add=True)

        plsc.subcore_barrier()
        pltpu.sync_copy(acc_sh.at[pl.ds(sid * rpw, rpw)],
                        out_hbm.at[cid, pl.ds(sid * rpw, rpw)])

    return pl.kernel(
        body,
        out_type=jax.ShapeDtypeStruct((NC, N, F), jnp.float32),
        mesh=_sc_mesh(),
        compiler_params=_sc_params(),
        scratch_types=[
            pltpu.VMEM((2, epw), jnp.int32),
            pltpu.VMEM((4, CHUNK, F), jnp.float32),
            pltpu.VMEM_SHARED((N, F), jnp.float32),
            pltpu.SemaphoreType.DMA,
            pltpu.SemaphoreType.DMA,
            pltpu.SemaphoreType.DMA,
            pltpu.SemaphoreType.DMA,
            pltpu.SemaphoreType.DMA,
            pltpu.SemaphoreType.DMA,
            pltpu.SemaphoreType.DMA,
            pltpu.SemaphoreType.DMA,
        ],
    )


def _dinv(degp_ref):
    d = degp_ref[:, 0:1] + degp_ref[:, 1:2] + 1.0
    return lax.rsqrt(d)


def _matmul_t(a, w_ref):
    return lax.dot_general(
        a, w_ref[...], dimension_numbers=(((1,), (1,)), ((), ())),
        preferred_element_type=jnp.float32, precision=lax.Precision.DEFAULT)


def _tc_mm1(x, W1, bn):
    N, D = x.shape
    H = W1.shape[0]

    def body(x_ref, w1_ref, o_ref):
        o_ref[...] = _matmul_t(x_ref[...], w1_ref)

    grid = (N // bn,)
    return pl.pallas_call(
        body,
        grid=grid,
        in_specs=[
            pl.BlockSpec((bn, D), lambda i: (i, 0)),
            pl.BlockSpec((H, D), lambda i: (0, 0)),
        ],
        out_specs=pl.BlockSpec((bn, H), lambda i: (i, 0)),
        out_shape=jax.ShapeDtypeStruct((N, H), jnp.float32),
    )(x, W1)


def _tc_scale(xw1, degt, bn):
    N, H = xw1.shape

    def body(xw_ref, degp_ref, o_ref):
        o_ref[...] = xw_ref[...] * _dinv(degp_ref)

    grid = (N // bn,)
    return pl.pallas_call(
        body,
        grid=grid,
        in_specs=[
            pl.BlockSpec((bn, H), lambda i: (i, 0)),
            pl.BlockSpec((bn, 2), lambda i: (i, 0)),
        ],
        out_specs=pl.BlockSpec((bn, H), lambda i: (i, 0)),
        out_shape=jax.ShapeDtypeStruct((N, H), jnp.float32),
    )(xw1, degt)


def _tc_mid(agg1, xs1, degt, W2, b1, bn, Fp):
    _, N, H = agg1.shape
    C = W2.shape[0]

    def body(a_ref, xs1_ref, degp_ref, w2_ref, b1_ref, o_ref):
        dinv = _dinv(degp_ref)
        s = a_ref[0] + a_ref[1] + xs1_ref[...]
        h = jnp.maximum(dinv * s + b1_ref[...], 0.0)
        xs2 = _matmul_t(h, w2_ref) * dinv
        if Fp > C:
            xs2 = jnp.concatenate(
                [xs2, jnp.zeros((xs2.shape[0], Fp - C), jnp.float32)], axis=1)
        o_ref[...] = xs2

    grid = (N // bn,)
    return pl.pallas_call(
        body,
        grid=grid,
        in_specs=[
            pl.BlockSpec((NC, bn, H), lambda i: (0, i, 0)),
            pl.BlockSpec((bn, H), lambda i: (i, 0)),
            pl.BlockSpec((bn, 2), lambda i: (i, 0)),
            pl.BlockSpec((C, H), lambda i: (0, 0)),
            pl.BlockSpec((1, H), lambda i: (0, 0)),
        ],
        out_specs=pl.BlockSpec((bn, Fp), lambda i: (i, 0)),
        out_shape=jax.ShapeDtypeStruct((N, Fp), jnp.float32),
    )(agg1, xs1, degt, W2, b1)


def _tc_final(agg2, xs2, degt, b2, bn):
    _, N, Fp = agg2.shape
    C = b2.shape[1]

    def body(a_ref, xs2_ref, degp_ref, b2_ref, o_ref):
        dinv = _dinv(degp_ref)
        s = a_ref[0] + a_ref[1] + xs2_ref[...]
        o_ref[...] = dinv * s[:, 0:C] + b2_ref[...]

    grid = (N // bn,)
    return pl.pallas_call(
        body,
        grid=grid,
        in_specs=[
            pl.BlockSpec((NC, bn, Fp), lambda i: (0, i, 0)),
            pl.BlockSpec((bn, Fp), lambda i: (i, 0)),
            pl.BlockSpec((bn, 2), lambda i: (i, 0)),
            pl.BlockSpec((1, C), lambda i: (0, 0)),
        ],
        out_specs=pl.BlockSpec((bn, C), lambda i: (i, 0)),
        out_shape=jax.ShapeDtypeStruct((N, C), jnp.float32),
    )(agg2, xs2, degt, b2)


def kernel(x, edge_index, W1, b1, W2, b2):
    N, D = x.shape
    H = W1.shape[0]
    C = W2.shape[0]
    E = edge_index.shape[1]
    Fp = _ceil_div(C, 8) * 8
    bn = 2000

    xw1 = _tc_mm1(x, W1, bn)
    degp = _deg_fn(N, E)(edge_index)
    degt = degp.T
    xs1 = _tc_scale(xw1, degt, bn)
    agg1 = _agg_fn(N, E, H)(xs1, edge_index)
    xs2 = _tc_mid(agg1, xs1, degt, W2, b1.reshape(1, H), bn, Fp)
    agg2 = _agg_fn(N, E, Fp)(xs2, edge_index)
    out = _tc_final(agg2, xs2, degt, b2.reshape(1, C), bn)
    return out


# speedup vs baseline: 1.0637x; 1.0068x over previous
import functools

import jax
import jax.numpy as jnp
from jax import lax
from jax.experimental import pallas as pl
from jax.experimental.pallas import tpu as pltpu
from jax.experimental.pallas import tpu_sc as plsc

NC = 2
NS = 16
NW = NC * NS
CHUNK = 128
DEGW = 16


def _ceil_div(a, b):
    return (a + b - 1) // b


def _sc_mesh():
    return plsc.VectorSubcoreMesh(core_axis_name="c", subcore_axis_name="s")


def _sc_params():
    return pltpu.CompilerParams(use_tc_tiling_on_sc=False)


@functools.lru_cache(maxsize=None)
def _deg_fn(N, E):
    assert E % NW == 0 and N % NS == 0
    epw = E // NW
    full = epw // CHUNK
    tail = epw % CHUNK
    nb = full // 3
    rem = full % 3
    stripe = (N // NS) & ~7
    last = N - (NS - 1) * stripe
    zlen = (last + 15) & ~15

    def body(ei_hbm, out_hbm, idx_v, ones_v, zero_v, acc_sh, sem_s):
        cid = lax.axis_index("c")
        sid = lax.axis_index("s")
        wid = cid * NS + sid

        @pl.loop(0, CHUNK, step=16)
        def _fill(i):
            ones_v[pl.ds(i, 16)] = jnp.ones((16,), jnp.float32)

        @pl.loop(0, zlen, step=16)
        def _zfill(i):
            zero_v[pl.ds(i, 16)] = jnp.zeros((16,), jnp.float32)

        @pl.when(sid < NS - 1)
        def _():
            pltpu.sync_copy(zero_v.at[pl.ds(0, stripe)],
                            acc_sh.at[pl.ds(sid * stripe, stripe)])

        @pl.when(sid == NS - 1)
        def _():
            pltpu.sync_copy(zero_v.at[pl.ds(0, last)],
                            acc_sh.at[pl.ds(sid * stripe, last)])

        pltpu.sync_copy(ei_hbm.at[1, pl.ds(wid * epw, epw)], idx_v.at[0])
        plsc.subcore_barrier()

        def fire(j):
            pltpu.async_copy(
                ones_v, acc_sh.at[idx_v.at[0, pl.ds(j * CHUNK, CHUNK)]],
                sem_s, add=True)

        def drain(j):
            pltpu.make_async_copy(
                ones_v, acc_sh.at[idx_v.at[0, pl.ds(j * CHUNK, CHUNK)]],
                sem_s).wait()

        @pl.loop(0, nb)
        def _edges(t):
            j = t * 3
            fire(j)
            fire(j + 1)
            fire(j + 2)

            @pl.when(t >= 1)
            def _():
                drain(j - 3)
                drain(j - 2)
                drain(j - 1)

        if nb > 0:
            drain(nb * 3 - 3)
            drain(nb * 3 - 2)
            drain(nb * 3 - 1)
        for u in range(rem):
            j = nb * 3 + u
            pltpu.sync_copy(
                ones_v, acc_sh.at[idx_v.at[0, pl.ds(j * CHUNK, CHUNK)]],
                add=True)
        if tail:
            pltpu.sync_copy(
                ones_v.at[pl.ds(0, tail)],
                acc_sh.at[idx_v.at[0, pl.ds(full * CHUNK, tail)]],
                add=True)

        plsc.subcore_barrier()

        @pl.when(sid < NS - 1)
        def _():
            pltpu.sync_copy(acc_sh.at[pl.ds(sid * stripe, stripe)],
                            out_hbm.at[cid, pl.ds(sid * stripe, stripe)])

        @pl.when(sid == NS - 1)
        def _():
            pltpu.sync_copy(acc_sh.at[pl.ds(sid * stripe, last)],
                            out_hbm.at[cid, pl.ds(sid * stripe, last)])

    return pl.kernel(
        body,
        out_type=jax.ShapeDtypeStruct((NC, N), jnp.float32),
        mesh=_sc_mesh(),
        compiler_params=_sc_params(),
        scratch_types=[
            pltpu.VMEM((1, epw), jnp.int32),
            pltpu.VMEM((CHUNK,), jnp.float32),
            pltpu.VMEM((zlen,), jnp.float32),
            pltpu.VMEM_SHARED((N,), jnp.float32),
            pltpu.SemaphoreType.DMA,
        ],
    )


@functools.lru_cache(maxsize=None)
def _agg_fn(N, E, F):
    assert E % NW == 0 and N % NS == 0 and F % 8 == 0
    epw = E // NW
    full = epw // CHUNK
    tail = epw % CHUNK
    nb = full // 4
    rem = full % 4
    loopn = nb * 4
    rpw = N // NS
    nz = _ceil_div(rpw, CHUNK)
    assert rpw % nz == 0
    zb = rpw // nz

    def body(src_hbm, ei_hbm, out_hbm, idx_v, rows_v, acc_sh,
             sg0, sg1, sg2, sg3, ss0, ss1, ss2, ss3):
        cid = lax.axis_index("c")
        sid = lax.axis_index("s")
        wid = cid * NS + sid
        sem_g = (sg0, sg1, sg2, sg3)
        sem_s = (ss0, ss1, ss2, ss3)

        pltpu.async_copy(ei_hbm.at[:, pl.ds(wid * epw, epw)], idx_v, sg0)

        zoffs = list(range(0, F - 15, 16))
        if F % 16:
            zoffs.append(F - 16)

        @pl.loop(0, CHUNK)
        def _fill(i):
            for j in zoffs:
                rows_v[0, i, pl.ds(j, 16)] = jnp.zeros((16,), jnp.float32)

        @pl.loop(0, nz)
        def _zero(k):
            pltpu.sync_copy(rows_v.at[0, pl.ds(0, zb)],
                            acc_sh.at[pl.ds(sid * rpw + k * zb, zb)])

        pltpu.make_async_copy(ei_hbm.at[:, pl.ds(wid * epw, epw)], idx_v,
                              sg0).wait()
        plsc.subcore_barrier()

        def gidx(j):
            return idx_v.at[0, pl.ds(j * CHUNK, CHUNK)]

        def sidx(j):
            return idx_v.at[1, pl.ds(j * CHUNK, CHUNK)]

        def fire_g(j, b):
            pltpu.async_copy(src_hbm.at[gidx(j)], rows_v.at[b], sem_g[b])

        def wait_g(j, b):
            pltpu.make_async_copy(src_hbm.at[gidx(j)], rows_v.at[b],
                                  sem_g[b]).wait()

        def fire_s(j, b):
            pltpu.async_copy(rows_v.at[b], acc_sh.at[sidx(j)], sem_s[b],
                             add=True)

        def wait_s(j, b):
            pltpu.make_async_copy(rows_v.at[b], acc_sh.at[sidx(j)],
                                  sem_s[b]).wait()

        if loopn > 0:
            fire_g(0, 0)
        if loopn > 1:
            fire_g(1, 1)

        @pl.loop(0, nb)
        def _edges(t):
            for u in range(4):
                j = t * 4 + u
                nx2 = (u + 2) % 4

                if u < 2:
                    @pl.when(j >= 2)
                    def _():
                        wait_s(j - 2, nx2)
                else:
                    wait_s(j - 2, nx2)

                if (loopn - 4 + u) + 2 < loopn:
                    fire_g(j + 2, nx2)
                else:
                    @pl.when(j + 2 < loopn)
                    def _():
                        fire_g(j + 2, nx2)
                wait_g(j, u)
                fire_s(j, u)

        if loopn > 1:
            wait_s(loopn - 2, (loopn - 2) % 4)
        if loopn > 0:
            wait_s(loopn - 1, (loopn - 1) % 4)
        for u in range(rem):
            j = loopn + u
            pltpu.sync_copy(src_hbm.at[gidx(j)], rows_v.at[0])
            pltpu.sync_copy(rows_v.at[0], acc_sh.at[sidx(j)], add=True)
        if tail:
            pltpu.sync_copy(
                src_hbm.at[idx_v.at[0, pl.ds(full * CHUNK, tail)]],
                rows_v.at[0, pl.ds(0, tail)])
            pltpu.sync_copy(
                rows_v.at[0, pl.ds(0, tail)],
                acc_sh.at[idx_v.at[1, pl.ds(full * CHUNK, tail)]],
                add=True)

        plsc.subcore_barrier()
        pltpu.sync_copy(acc_sh.at[pl.ds(sid * rpw, rpw)],
                        out_hbm.at[cid, pl.ds(sid * rpw, rpw)])

    return pl.kernel(
        body,
        out_type=jax.ShapeDtypeStruct((NC, N, F), jnp.float32),
        mesh=_sc_mesh(),
        compiler_params=_sc_params(),
        scratch_types=[
            pltpu.VMEM((2, epw), jnp.int32),
            pltpu.VMEM((4, CHUNK, F), jnp.float32),
            pltpu.VMEM_SHARED((N, F), jnp.float32),
            pltpu.SemaphoreType.DMA,
            pltpu.SemaphoreType.DMA,
            pltpu.SemaphoreType.DMA,
            pltpu.SemaphoreType.DMA,
            pltpu.SemaphoreType.DMA,
            pltpu.SemaphoreType.DMA,
            pltpu.SemaphoreType.DMA,
            pltpu.SemaphoreType.DMA,
        ],
    )


def _dinv(degp_ref):
    d = degp_ref[:, 0:1] + degp_ref[:, 1:2] + 1.0
    return lax.rsqrt(d)


def _matmul_t(a, w_ref):
    return lax.dot_general(
        a, w_ref[...], dimension_numbers=(((1,), (1,)), ((), ())),
        preferred_element_type=jnp.float32, precision=lax.Precision.DEFAULT)


def _tc_mm1(x, W1, bn):
    N, D = x.shape
    H = W1.shape[0]

    def body(x_ref, w1_ref, o_ref):
        o_ref[...] = _matmul_t(x_ref[...], w1_ref)

    grid = (N // bn,)
    return pl.pallas_call(
        body,
        grid=grid,
        in_specs=[
            pl.BlockSpec((bn, D), lambda i: (i, 0)),
            pl.BlockSpec((H, D), lambda i: (0, 0)),
        ],
        out_specs=pl.BlockSpec((bn, H), lambda i: (i, 0)),
        out_shape=jax.ShapeDtypeStruct((N, H), jnp.float32),
    )(x, W1)


def _tc_scale(xw1, degt, bn):
    N, H = xw1.shape

    def body(xw_ref, degp_ref, o_ref):
        o_ref[...] = xw_ref[...] * _dinv(degp_ref)

    grid = (N // bn,)
    return pl.pallas_call(
        body,
        grid=grid,
        in_specs=[
            pl.BlockSpec((bn, H), lambda i: (i, 0)),
            pl.BlockSpec((bn, 2), lambda i: (i, 0)),
        ],
        out_specs=pl.BlockSpec((bn, H), lambda i: (i, 0)),
        out_shape=jax.ShapeDtypeStruct((N, H), jnp.float32),
    )(xw1, degt)


def _tc_mid(agg1, xs1, degt, W2, b1, bn, Fp):
    _, N, H = agg1.shape
    C = W2.shape[0]

    def body(a_ref, xs1_ref, degp_ref, w2_ref, b1_ref, o_ref):
        dinv = _dinv(degp_ref)
        s = a_ref[0] + a_ref[1] + xs1_ref[...]
        h = jnp.maximum(dinv * s + b1_ref[...], 0.0)
        xs2 = _matmul_t(h, w2_ref) * dinv
        if Fp > C:
            xs2 = jnp.concatenate(
                [xs2, jnp.zeros((xs2.shape[0], Fp - C), jnp.float32)], axis=1)
        o_ref[...] = xs2

    grid = (N // bn,)
    return pl.pallas_call(
        body,
        grid=grid,
        in_specs=[
            pl.BlockSpec((NC, bn, H), lambda i: (0, i, 0)),
            pl.BlockSpec((bn, H), lambda i: (i, 0)),
            pl.BlockSpec((bn, 2), lambda i: (i, 0)),
            pl.BlockSpec((C, H), lambda i: (0, 0)),
            pl.BlockSpec((1, H), lambda i: (0, 0)),
        ],
        out_specs=pl.BlockSpec((bn, Fp), lambda i: (i, 0)),
        out_shape=jax.ShapeDtypeStruct((N, Fp), jnp.float32),
        compiler_params=pltpu.CompilerParams(
            allow_input_fusion=[True, True, True, True, True]),
    )(agg1, xs1, degt, W2, b1)


def _tc_final(agg2, xs2, degt, b2, bn):
    _, N, Fp = agg2.shape
    C = b2.shape[1]

    def body(a_ref, xs2_ref, degp_ref, b2_ref, o_ref):
        dinv = _dinv(degp_ref)
        s = a_ref[0] + a_ref[1] + xs2_ref[...]
        o_ref[...] = dinv * s[:, 0:C] + b2_ref[...]

    grid = (N // bn,)
    return pl.pallas_call(
        body,
        grid=grid,
        in_specs=[
            pl.BlockSpec((NC, bn, Fp), lambda i: (0, i, 0)),
            pl.BlockSpec((bn, Fp), lambda i: (i, 0)),
            pl.BlockSpec((bn, 2), lambda i: (i, 0)),
            pl.BlockSpec((1, C), lambda i: (0, 0)),
        ],
        out_specs=pl.BlockSpec((bn, C), lambda i: (i, 0)),
        out_shape=jax.ShapeDtypeStruct((N, C), jnp.float32),
        compiler_params=pltpu.CompilerParams(
            allow_input_fusion=[True, True, True, True]),
    )(agg2, xs2, degt, b2)


def kernel(x, edge_index, W1, b1, W2, b2):
    N, D = x.shape
    H = W1.shape[0]
    C = W2.shape[0]
    E = edge_index.shape[1]
    Fp = _ceil_div(C, 8) * 8
    bn = 2000

    xw1 = _tc_mm1(x, W1, bn)
    degp = _deg_fn(N, E)(edge_index)
    degt = degp.T
    xs1 = _tc_scale(xw1, degt, bn)
    agg1 = _agg_fn(N, E, H)(xs1, edge_index)
    xs2 = _tc_mid(agg1, xs1, degt, W2, b1.reshape(1, H), bn, Fp)
    agg2 = _agg_fn(N, E, Fp)(xs2, edge_index)
    out = _tc_final(agg2, xs2, degt, b2.reshape(1, C), bn)
    return out


# speedup vs baseline: 1.0649x; 1.0011x over previous
import functools

import jax
import jax.numpy as jnp
from jax import lax
from jax.experimental import pallas as pl
from jax.experimental.pallas import tpu as pltpu
from jax.experimental.pallas import tpu_sc as plsc

NC = 2
NS = 16
NW = NC * NS
CHUNK = 128


def _ceil_div(a, b):
    return (a + b - 1) // b


def _sc_mesh():
    return plsc.VectorSubcoreMesh(core_axis_name="c", subcore_axis_name="s")


def _sc_params():
    return pltpu.CompilerParams(use_tc_tiling_on_sc=False)


@functools.lru_cache(maxsize=None)
def _deg_fn(N, E):
    assert E % NW == 0 and N % NS == 0
    epw = E // NW
    full = epw // CHUNK
    tail = epw % CHUNK
    nb = full // 3
    rem = full % 3
    stripe = (N // NS) & ~7
    last = N - (NS - 1) * stripe
    zlen = (last + 15) & ~15

    def body(ei_hbm, out_hbm, idx_v, ones_v, zero_v, acc_sh, sem_s):
        cid = lax.axis_index("c")
        sid = lax.axis_index("s")
        wid = cid * NS + sid

        @pl.loop(0, CHUNK, step=16)
        def _fill(i):
            ones_v[pl.ds(i, 16)] = jnp.ones((16,), jnp.float32)

        @pl.loop(0, zlen, step=16)
        def _zfill(i):
            zero_v[pl.ds(i, 16)] = jnp.zeros((16,), jnp.float32)

        @pl.when(sid < NS - 1)
        def _():
            pltpu.sync_copy(zero_v.at[pl.ds(0, stripe)],
                            acc_sh.at[pl.ds(sid * stripe, stripe)])

        @pl.when(sid == NS - 1)
        def _():
            pltpu.sync_copy(zero_v.at[pl.ds(0, last)],
                            acc_sh.at[pl.ds(sid * stripe, last)])

        pltpu.sync_copy(ei_hbm.at[1, pl.ds(wid * epw, epw)], idx_v.at[0])
        plsc.subcore_barrier()

        def fire(j):
            pltpu.async_copy(
                ones_v, acc_sh.at[idx_v.at[0, pl.ds(j * CHUNK, CHUNK)]],
                sem_s, add=True)

        def drain(j):
            pltpu.make_async_copy(
                ones_v, acc_sh.at[idx_v.at[0, pl.ds(j * CHUNK, CHUNK)]],
                sem_s).wait()

        @pl.loop(0, nb)
        def _edges(t):
            j = t * 3
            fire(j)
            fire(j + 1)
            fire(j + 2)

            @pl.when(t >= 1)
            def _():
                drain(j - 3)
                drain(j - 2)
                drain(j - 1)

        if nb > 0:
            drain(nb * 3 - 3)
            drain(nb * 3 - 2)
            drain(nb * 3 - 1)
        for u in range(rem):
            j = nb * 3 + u
            pltpu.sync_copy(
                ones_v, acc_sh.at[idx_v.at[0, pl.ds(j * CHUNK, CHUNK)]],
                add=True)
        if tail:
            pltpu.sync_copy(
                ones_v.at[pl.ds(0, tail)],
                acc_sh.at[idx_v.at[0, pl.ds(full * CHUNK, tail)]],
                add=True)

        plsc.subcore_barrier()

        @pl.when(sid < NS - 1)
        def _():
            pltpu.sync_copy(acc_sh.at[pl.ds(sid * stripe, stripe)],
                            out_hbm.at[cid, pl.ds(sid * stripe, stripe)])

        @pl.when(sid == NS - 1)
        def _():
            pltpu.sync_copy(acc_sh.at[pl.ds(sid * stripe, last)],
                            out_hbm.at[cid, pl.ds(sid * stripe, last)])

    return pl.kernel(
        body,
        out_type=jax.ShapeDtypeStruct((NC, N), jnp.float32),
        mesh=_sc_mesh(),
        compiler_params=_sc_params(),
        scratch_types=[
            pltpu.VMEM((1, epw), jnp.int32),
            pltpu.VMEM((CHUNK,), jnp.float32),
            pltpu.VMEM((zlen,), jnp.float32),
            pltpu.VMEM_SHARED((N,), jnp.float32),
            pltpu.SemaphoreType.DMA,
        ],
    )


@functools.lru_cache(maxsize=None)
def _agg_fn(N, E, F):
    assert E % NW == 0 and N % NS == 0 and F % 8 == 0
    epw = E // NW
    full = epw // CHUNK
    tail = epw % CHUNK
    nb = full // 4
    rem = full % 4
    loopn = nb * 4
    rpw = N // NS
    nz = _ceil_div(rpw, CHUNK)
    assert rpw % nz == 0
    zb = rpw // nz

    def body(src_hbm, ei_hbm, out_hbm, idx_v, rows_v, acc_sh,
             sg0, sg1, sg2, sg3, ss0, ss1, ss2, ss3):
        cid = lax.axis_index("c")
        sid = lax.axis_index("s")
        wid = cid * NS + sid
        sem_g = (sg0, sg1, sg2, sg3)
        sem_s = (ss0, ss1, ss2, ss3)

        pltpu.async_copy(ei_hbm.at[:, pl.ds(wid * epw, epw)], idx_v, sg0)

        zoffs = list(range(0, F - 15, 16))
        if F % 16:
            zoffs.append(F - 16)

        @pl.loop(0, CHUNK)
        def _fill(i):
            for j in zoffs:
                rows_v[0, i, pl.ds(j, 16)] = jnp.zeros((16,), jnp.float32)

        @pl.loop(0, nz)
        def _zero(k):
            pltpu.sync_copy(rows_v.at[0, pl.ds(0, zb)],
                            acc_sh.at[pl.ds(sid * rpw + k * zb, zb)])

        pltpu.make_async_copy(ei_hbm.at[:, pl.ds(wid * epw, epw)], idx_v,
                              sg0).wait()
        plsc.subcore_barrier()

        def gidx(j):
            return idx_v.at[0, pl.ds(j * CHUNK, CHUNK)]

        def sidx(j):
            return idx_v.at[1, pl.ds(j * CHUNK, CHUNK)]

        def fire_g(j, b):
            pltpu.async_copy(src_hbm.at[gidx(j)], rows_v.at[b], sem_g[b])

        def wait_g(j, b):
            pltpu.make_async_copy(src_hbm.at[gidx(j)], rows_v.at[b],
                                  sem_g[b]).wait()

        def fire_s(j, b):
            pltpu.async_copy(rows_v.at[b], acc_sh.at[sidx(j)], sem_s[b],
                             add=True)

        def wait_s(j, b):
            pltpu.make_async_copy(rows_v.at[b], acc_sh.at[sidx(j)],
                                  sem_s[b]).wait()

        if loopn > 0:
            fire_g(0, 0)
        if loopn > 1:
            fire_g(1, 1)

        @pl.loop(0, nb)
        def _edges(t):
            for u in range(4):
                j = t * 4 + u
                nx2 = (u + 2) % 4

                if u < 2:
                    @pl.when(j >= 2)
                    def _():
                        wait_s(j - 2, nx2)
                else:
                    wait_s(j - 2, nx2)

                if (loopn - 4 + u) + 2 < loopn:
                    fire_g(j + 2, nx2)
                else:
                    @pl.when(j + 2 < loopn)
                    def _():
                        fire_g(j + 2, nx2)
                wait_g(j, u)
                fire_s(j, u)

        if loopn > 1:
            wait_s(loopn - 2, (loopn - 2) % 4)
        if loopn > 0:
            wait_s(loopn - 1, (loopn - 1) % 4)
        for u in range(rem):
            j = loopn + u
            pltpu.sync_copy(src_hbm.at[gidx(j)], rows_v.at[0])
            pltpu.sync_copy(rows_v.at[0], acc_sh.at[sidx(j)], add=True)
        if tail:
            pltpu.sync_copy(
                src_hbm.at[idx_v.at[0, pl.ds(full * CHUNK, tail)]],
                rows_v.at[0, pl.ds(0, tail)])
            pltpu.sync_copy(
                rows_v.at[0, pl.ds(0, tail)],
                acc_sh.at[idx_v.at[1, pl.ds(full * CHUNK, tail)]],
                add=True)

        plsc.subcore_barrier()
        pltpu.sync_copy(acc_sh.at[pl.ds(sid * rpw, rpw)],
                        out_hbm.at[cid, pl.ds(sid * rpw, rpw)])

    return pl.kernel(
        body,
        out_type=jax.ShapeDtypeStruct((NC, N, F), jnp.float32),
        mesh=_sc_mesh(),
        compiler_params=_sc_params(),
        scratch_types=[
            pltpu.VMEM((2, epw), jnp.int32),
            pltpu.VMEM((4, CHUNK, F), jnp.float32),
            pltpu.VMEM_SHARED((N, F), jnp.float32),
            pltpu.SemaphoreType.DMA,
            pltpu.SemaphoreType.DMA,
            pltpu.SemaphoreType.DMA,
            pltpu.SemaphoreType.DMA,
            pltpu.SemaphoreType.DMA,
            pltpu.SemaphoreType.DMA,
            pltpu.SemaphoreType.DMA,
            pltpu.SemaphoreType.DMA,
        ],
    )


def _dinv(degp_ref):
    d = degp_ref[:, 0:1] + degp_ref[:, 1:2] + 1.0
    return lax.rsqrt(d)


def _matmul_t(a, w_ref):
    return lax.dot_general(
        a, w_ref[...], dimension_numbers=(((1,), (1,)), ((), ())),
        preferred_element_type=jnp.float32, precision=lax.Precision.DEFAULT)


def _tc_mm1(x, W1, bn):
    N, D = x.shape
    H = W1.shape[0]

    def body(x_ref, w1_ref, o_ref):
        o_ref[...] = _matmul_t(x_ref[...], w1_ref)

    grid = (N // bn,)
    return pl.pallas_call(
        body,
        grid=grid,
        in_specs=[
            pl.BlockSpec((bn, D), lambda i: (i, 0)),
            pl.BlockSpec((H, D), lambda i: (0, 0)),
        ],
        out_specs=pl.BlockSpec((bn, H), lambda i: (i, 0)),
        out_shape=jax.ShapeDtypeStruct((N, H), jnp.float32),
    )(x, W1)


def _tc_scale(xw1, degt, bn):
    N, H = xw1.shape

    def body(xw_ref, degp_ref, o_ref):
        o_ref[...] = xw_ref[...] * _dinv(degp_ref)

    grid = (N // bn,)
    return pl.pallas_call(
        body,
        grid=grid,
        in_specs=[
            pl.BlockSpec((bn, H), lambda i: (i, 0)),
            pl.BlockSpec((bn, 2), lambda i: (i, 0)),
        ],
        out_specs=pl.BlockSpec((bn, H), lambda i: (i, 0)),
        out_shape=jax.ShapeDtypeStruct((N, H), jnp.float32),
    )(xw1, degt)


def _tc_mid(agg1, xs1, degt, W2, b1, bn, Fp):
    _, N, H = agg1.shape
    C = W2.shape[0]

    def body(a_ref, xs1_ref, degp_ref, w2_ref, b1_ref, o_ref):
        dinv = _dinv(degp_ref)
        s = a_ref[0] + a_ref[1] + xs1_ref[...]
        h = jnp.maximum(dinv * s + b1_ref[...], 0.0)
        xs2 = _matmul_t(h, w2_ref) * dinv
        if Fp > C:
            xs2 = jnp.concatenate(
                [xs2, jnp.zeros((xs2.shape[0], Fp - C), jnp.float32)], axis=1)
        o_ref[...] = xs2

    grid = (N // bn,)
    return pl.pallas_call(
        body,
        grid=grid,
        in_specs=[
            pl.BlockSpec((NC, bn, H), lambda i: (0, i, 0)),
            pl.BlockSpec((bn, H), lambda i: (i, 0)),
            pl.BlockSpec((bn, 2), lambda i: (i, 0)),
            pl.BlockSpec((C, H), lambda i: (0, 0)),
            pl.BlockSpec((1, H), lambda i: (0, 0)),
        ],
        out_specs=pl.BlockSpec((bn, Fp), lambda i: (i, 0)),
        out_shape=jax.ShapeDtypeStruct((N, Fp), jnp.float32),
        compiler_params=pltpu.CompilerParams(
            allow_input_fusion=[True, True, True, True, True]),
    )(agg1, xs1, degt, W2, b1)


def _tc_final(agg2, xs2, degt, b2, bn):
    _, N, Fp = agg2.shape
    C = b2.shape[1]

    def body(a_ref, xs2_ref, degp_ref, b2_ref, o_ref):
        dinv = _dinv(degp_ref)
        s = a_ref[0] + a_ref[1] + xs2_ref[...]
        o_ref[...] = dinv * s[:, 0:C] + b2_ref[...]

    grid = (N // bn,)
    return pl.pallas_call(
        body,
        grid=grid,
        in_specs=[
            pl.BlockSpec((NC, bn, Fp), lambda i: (0, i, 0)),
            pl.BlockSpec((bn, Fp), lambda i: (i, 0)),
            pl.BlockSpec((bn, 2), lambda i: (i, 0)),
            pl.BlockSpec((1, C), lambda i: (0, 0)),
        ],
        out_specs=pl.BlockSpec((bn, C), lambda i: (i, 0)),
        out_shape=jax.ShapeDtypeStruct((N, C), jnp.float32),
        compiler_params=pltpu.CompilerParams(
            allow_input_fusion=[True, True, True, True]),
    )(agg2, xs2, degt, b2)


def kernel(x, edge_index, W1, b1, W2, b2):
    N, D = x.shape
    H = W1.shape[0]
    C = W2.shape[0]
    E = edge_index.shape[1]
    Fp = _ceil_div(C, 8) * 8
    bn = 2000

    xw1 = _tc_mm1(x, W1, bn)
    degp = _deg_fn(N, E)(edge_index)
    degt = degp.T
    xs1 = _tc_scale(xw1, degt, bn)
    agg1 = _agg_fn(N, E, H)(xs1, edge_index)
    xs2 = _tc_mid(agg1, xs1, degt, W2, b1.reshape(1, H), bn, Fp)
    agg2 = _agg_fn(N, E, Fp)(xs2, edge_index)
    out = _tc_final(agg2, xs2, degt, b2.reshape(1, C), bn)
    return out
